# baseline (speedup 1.0000x reference)
import functools

import numpy as np
import jax
import jax.numpy as jnp
from jax.experimental import pallas as pl
from jax.experimental.pallas import tpu as pltpu

C = 96
K = 3
NCH = C * K * K
TOPK = C * K
NSORT = 1024
H = 224
W = 224
INT_MIN = np.int32(-2**31)


def _row_kernel(x0_ref, x1_ref, x2_ref, wr_ref, wk_ref, wm_ref, o_ref):
    wr = wr_ref[...].astype(jnp.float32)
    wk = wk_ref[...].astype(jnp.float32)

    zr = None
    zk = None
    for dy, xref in enumerate((x0_ref, x1_ref, x2_ref)):
        xrow = xref[0].astype(jnp.float32)
        xt = jnp.concatenate([xrow] * (K * K), axis=0)
        for dx in range(K):
            t = dy * K + dx
            xsh = xt[:, dx:dx + W]
            cr = wr[:, t:t + 1] * xsh
            ck = wk[:, t:t + 1] * xsh
            zr = cr if zr is None else zr + cr
            zk = ck if zk is None else zk + ck

    m = jnp.max(zr, axis=0, keepdims=True)
    e = jnp.exp(zr - m)
    s = jnp.sum(e, axis=0, keepdims=True)
    r = e * (1.0 / s)

    keys = jax.lax.bitcast_convert_type(r, jnp.int32)

    row = jax.lax.broadcasted_iota(jnp.int32, (NCH, W), 0)
    idx = (row % C) * (K * K) + row // C

    pad = NSORT - NCH
    keys = jnp.concatenate(
        [keys, jnp.full((pad, W), INT_MIN, jnp.int32)], axis=0)
    idx = jnp.concatenate(
        [idx, jnp.full((pad, W), NCH, jnp.int32)], axis=0)
    kv = jnp.concatenate([zk, jnp.zeros((pad, W), jnp.float32)], axis=0)

    riota = jax.lax.broadcasted_iota(jnp.int32, (NSORT, 1), 0)

    for kk in [2 << t for t in range(10)]:
        jj = kk // 2
        while jj >= 1:
            if jj >= 8:
                mrows = NSORT // (2 * jj)
                k4 = keys.reshape(mrows, 2, jj, W)
                i4 = idx.reshape(mrows, 2, jj, W)
                v4 = kv.reshape(mrows, 2, jj, W)
                ka, kb = k4[:, 0], k4[:, 1]
                ia, ib = i4[:, 0], i4[:, 1]
                va, vb = v4[:, 0], v4[:, 1]
                sw = (kb > ka) | ((kb == ka) & (ib < ia))
                miota = jax.lax.broadcasted_iota(jnp.int32, (mrows, 1, 1), 0)
                dirc = ((miota * (2 * jj)) & kk) != 0
                swap = sw ^ dirc
                nka = jnp.where(swap, kb, ka)
                nkb = jnp.where(swap, ka, kb)
                nia = jnp.where(swap, ib, ia)
                nib = jnp.where(swap, ia, ib)
                nva = jnp.where(swap, vb, va)
                nvb = jnp.where(swap, va, vb)
                keys = jnp.concatenate(
                    [nka[:, None], nkb[:, None]], axis=1).reshape(NSORT, W)
                idx = jnp.concatenate(
                    [nia[:, None], nib[:, None]], axis=1).reshape(NSORT, W)
                kv = jnp.concatenate(
                    [nva[:, None], nvb[:, None]], axis=1).reshape(NSORT, W)
            else:
                bitj = (riota & jj) != 0
                cmask = bitj ^ ((riota & kk) != 0)
                kp = jnp.where(bitj, pltpu.roll(keys, jj, 0),
                               pltpu.roll(keys, NSORT - jj, 0))
                ip = jnp.where(bitj, pltpu.roll(idx, jj, 0),
                               pltpu.roll(idx, NSORT - jj, 0))
                vp = jnp.where(bitj, pltpu.roll(kv, jj, 0),
                               pltpu.roll(kv, NSORT - jj, 0))
                s1 = (kp > keys) | ((kp == keys) & (ip < idx))
                swap = s1 ^ cmask
                keys = jnp.where(swap, kp, keys)
                idx = jnp.where(swap, ip, idx)
                kv = jnp.where(swap, vp, kv)
            jj //= 2

    rtop = jax.lax.bitcast_convert_type(keys[:TOPK], jnp.float32)
    ktop = kv[:TOPK]
    sv = jnp.concatenate([rtop, ktop], axis=0)

    y = jax.lax.dot_general(
        wm_ref[...], sv.astype(jnp.bfloat16),
        (((1,), (0,)), ((), ())),
        preferred_element_type=jnp.float32)
    o_ref[0, :, :] = y


def kernel(x, Wr, Wk, Wc, bc):
    N = x.shape[0]
    rb = lambda a: a.astype(jnp.bfloat16)
    xp = rb(jnp.pad(jnp.moveaxis(x[0], 0, 1),
                    ((1, 1), (0, 0), (1, 1))))
    wr2 = rb(jnp.transpose(Wr.reshape(C, K * K, K, K),
                           (1, 0, 2, 3)).reshape(NCH, K * K))
    wk2 = rb(jnp.transpose(Wk.reshape(C, K * K, K, K),
                           (1, 0, 2, 3)).reshape(NCH, K * K))
    wm16 = Wc[:, :, 0, 0].astype(jnp.bfloat16)

    rowspec = lambda off: pl.BlockSpec(
        (1, C, W + 2), lambda i, off=off: (i + off, 0, 0))
    y = pl.pallas_call(
        _row_kernel,
        grid=(H,),
        in_specs=[
            rowspec(0), rowspec(1), rowspec(2),
            pl.BlockSpec((NCH, K * K), lambda i: (0, 0)),
            pl.BlockSpec((NCH, K * K), lambda i: (0, 0)),
            pl.BlockSpec((C, 2 * TOPK), lambda i: (0, 0)),
        ],
        out_specs=pl.BlockSpec((1, C, W), lambda i: (i, 0, 0)),
        out_shape=jax.ShapeDtypeStruct((H, C, W), jnp.float32),
    )(xp, xp, xp, wr2, wk2, wm16)
    return jnp.moveaxis(y, 0, 1)[None] + bc[None, :, None, None]


# speedup vs baseline: 1.3913x; 1.3913x over previous
import functools

import numpy as np
import jax
import jax.numpy as jnp
from jax.experimental import pallas as pl
from jax.experimental.pallas import tpu as pltpu

C = 96
K = 3
NCH = C * K * K
TOPK = C * K
NSORT = 1024
H = 224
W = 224
RB = 4
WB = RB * W
INT_MIN = np.int32(-2**31)


def _row_kernel(x0_ref, x1_ref, x2_ref, x3_ref, x4_ref, x5_ref,
                wr_ref, wk_ref, wm_ref, o_ref):
    xrefs = (x0_ref, x1_ref, x2_ref, x3_ref, x4_ref, x5_ref)
    wr = wr_ref[...].astype(jnp.float32)
    wk = wk_ref[...].astype(jnp.float32)

    zr_rows = []
    zk_rows = []
    for rr in range(RB):
        zr = None
        zk = None
        for dy in range(K):
            xrow = xrefs[rr + dy][0].astype(jnp.float32)
            xt = jnp.concatenate([xrow] * (K * K), axis=0)
            for dx in range(K):
                t = dy * K + dx
                xsh = xt[:, dx:dx + W]
                cr = wr[:, t:t + 1] * xsh
                ck = wk[:, t:t + 1] * xsh
                zr = cr if zr is None else zr + cr
                zk = ck if zk is None else zk + ck
        zr_rows.append(zr)
        zk_rows.append(zk)
    zr = jnp.concatenate(zr_rows, axis=1)
    zk = jnp.concatenate(zk_rows, axis=1)

    m = jnp.max(zr, axis=0, keepdims=True)
    e = jnp.exp(zr - m)
    s = jnp.sum(e, axis=0, keepdims=True)
    r = e * (1.0 / s)

    keys = jax.lax.bitcast_convert_type(r, jnp.int32)

    row = jax.lax.broadcasted_iota(jnp.int32, (NCH, WB), 0)
    idx = (row % C) * (K * K) + row // C

    pad = NSORT - NCH
    keys = jnp.concatenate(
        [keys, jnp.full((pad, WB), INT_MIN, jnp.int32)], axis=0)
    idx = jnp.concatenate(
        [idx, jnp.full((pad, WB), NCH, jnp.int32)], axis=0)
    kv = jnp.concatenate([zk, jnp.zeros((pad, WB), jnp.float32)], axis=0)

    riota = jax.lax.broadcasted_iota(jnp.int32, (NSORT, 1), 0)

    for kk in [2 << t for t in range(10)]:
        jj = kk // 2
        while jj >= 1:
            if jj >= 8:
                mrows = NSORT // (2 * jj)
                k4 = keys.reshape(mrows, 2, jj, WB)
                i4 = idx.reshape(mrows, 2, jj, WB)
                v4 = kv.reshape(mrows, 2, jj, WB)
                ka, kb = k4[:, 0], k4[:, 1]
                ia, ib = i4[:, 0], i4[:, 1]
                va, vb = v4[:, 0], v4[:, 1]
                sw = (kb > ka) | ((kb == ka) & (ib < ia))
                miota = jax.lax.broadcasted_iota(jnp.int32, (mrows, 1, 1), 0)
                dirc = ((miota * (2 * jj)) & kk) != 0
                swap = sw ^ dirc
                nka = jnp.where(swap, kb, ka)
                nkb = jnp.where(swap, ka, kb)
                nia = jnp.where(swap, ib, ia)
                nib = jnp.where(swap, ia, ib)
                nva = jnp.where(swap, vb, va)
                nvb = jnp.where(swap, va, vb)
                keys = jnp.concatenate(
                    [nka[:, None], nkb[:, None]], axis=1).reshape(NSORT, WB)
                idx = jnp.concatenate(
                    [nia[:, None], nib[:, None]], axis=1).reshape(NSORT, WB)
                kv = jnp.concatenate(
                    [nva[:, None], nvb[:, None]], axis=1).reshape(NSORT, WB)
            else:
                bitj = (riota & jj) != 0
                cmask = bitj ^ ((riota & kk) != 0)
                kp = jnp.where(bitj, pltpu.roll(keys, jj, 0),
                               pltpu.roll(keys, NSORT - jj, 0))
                ip = jnp.where(bitj, pltpu.roll(idx, jj, 0),
                               pltpu.roll(idx, NSORT - jj, 0))
                vp = jnp.where(bitj, pltpu.roll(kv, jj, 0),
                               pltpu.roll(kv, NSORT - jj, 0))
                s1 = (kp > keys) | ((kp == keys) & (ip < idx))
                swap = s1 ^ cmask
                keys = jnp.where(swap, kp, keys)
                idx = jnp.where(swap, ip, idx)
                kv = jnp.where(swap, vp, kv)
            jj //= 2

    rtop = jax.lax.bitcast_convert_type(keys[:TOPK], jnp.float32)
    ktop = kv[:TOPK]
    sv = jnp.concatenate([rtop, ktop], axis=0)

    y = jax.lax.dot_general(
        wm_ref[...], sv.astype(jnp.bfloat16),
        (((1,), (0,)), ((), ())),
        preferred_element_type=jnp.float32)
    o_ref[0, :, :] = y


def kernel(x, Wr, Wk, Wc, bc):
    rb = lambda a: a.astype(jnp.bfloat16)
    xp = rb(jnp.pad(jnp.moveaxis(x[0], 0, 1),
                    ((1, 1), (0, 0), (1, 1))))
    wr2 = rb(jnp.transpose(Wr.reshape(C, K * K, K, K),
                           (1, 0, 2, 3)).reshape(NCH, K * K))
    wk2 = rb(jnp.transpose(Wk.reshape(C, K * K, K, K),
                           (1, 0, 2, 3)).reshape(NCH, K * K))
    wm16 = Wc[:, :, 0, 0].astype(jnp.bfloat16)

    rowspec = lambda off: pl.BlockSpec(
        (1, C, W + 2), lambda i, off=off: (RB * i + off, 0, 0))
    y = pl.pallas_call(
        _row_kernel,
        grid=(H // RB,),
        in_specs=[
            rowspec(0), rowspec(1), rowspec(2),
            rowspec(3), rowspec(4), rowspec(5),
            pl.BlockSpec((NCH, K * K), lambda i: (0, 0)),
            pl.BlockSpec((NCH, K * K), lambda i: (0, 0)),
            pl.BlockSpec((C, 2 * TOPK), lambda i: (0, 0)),
        ],
        out_specs=pl.BlockSpec((1, C, WB), lambda i: (i, 0, 0)),
        out_shape=jax.ShapeDtypeStruct((H // RB, C, WB), jnp.float32),
    )(xp, xp, xp, xp, xp, xp, wr2, wk2, wm16)
    y = y.reshape(H // RB, C, RB, W).transpose(1, 0, 2, 3).reshape(C, H, W)
    return (y + bc[:, None, None])[None]


# speedup vs baseline: 1.4029x; 1.0083x over previous
import functools

import numpy as np
import jax
import jax.numpy as jnp
from jax.experimental import pallas as pl
from jax.experimental.pallas import tpu as pltpu

C = 96
K = 3
NCH = C * K * K
TOPK = C * K
NSORT = 1024
H = 224
W = 224
RB = 4
WB = RB * W
INT_MIN = np.int32(-2**31)


def _row_kernel(x0_ref, x1_ref, x2_ref, x3_ref, x4_ref, x5_ref,
                wr_ref, wk_ref, wm_ref, o_ref):
    xrefs = (x0_ref, x1_ref, x2_ref, x3_ref, x4_ref, x5_ref)
    wr = wr_ref[...].astype(jnp.float32)
    wk = wk_ref[...].astype(jnp.float32)

    zr_rows = []
    zk_rows = []
    for rr in range(RB):
        zr = None
        zk = None
        for dy in range(K):
            xrow = xrefs[rr + dy][0].astype(jnp.float32)
            xt = jnp.concatenate([xrow] * (K * K), axis=0)
            for dx in range(K):
                t = dy * K + dx
                xsh = xt[:, dx:dx + W]
                cr = wr[:, t:t + 1] * xsh
                ck = wk[:, t:t + 1] * xsh
                zr = cr if zr is None else zr + cr
                zk = ck if zk is None else zk + ck
        zr_rows.append(zr)
        zk_rows.append(zk)
    zr = jnp.concatenate(zr_rows, axis=1)
    zk = jnp.concatenate(zk_rows, axis=1)

    m = jnp.max(zr, axis=0, keepdims=True)
    e = jnp.exp(zr - m)
    s = jnp.sum(e, axis=0, keepdims=True)
    r = e * (1.0 / s)

    keys = jax.lax.bitcast_convert_type(r, jnp.int32)

    row = jax.lax.broadcasted_iota(jnp.int32, (NCH, WB), 0)
    idx = (row % C) * (K * K) + row // C

    pad = NSORT - NCH
    keys = jnp.concatenate(
        [keys, jnp.full((pad, WB), INT_MIN, jnp.int32)], axis=0)
    idx = jnp.concatenate(
        [idx, jnp.full((pad, WB), NCH, jnp.int32)], axis=0)
    kv = jnp.concatenate([zk, jnp.zeros((pad, WB), jnp.float32)], axis=0)

    def _stage(keys, idx, kv, kk, jj, n):
        if jj >= 8:
            mrows = n // (2 * jj)
            k4 = keys.reshape(mrows, 2, jj, WB)
            i4 = idx.reshape(mrows, 2, jj, WB)
            v4 = kv.reshape(mrows, 2, jj, WB)
            ka, kb = k4[:, 0], k4[:, 1]
            ia, ib = i4[:, 0], i4[:, 1]
            va, vb = v4[:, 0], v4[:, 1]
            sw = (kb > ka) | ((kb == ka) & (ib < ia))
            miota = jax.lax.broadcasted_iota(jnp.int32, (mrows, 1, 1), 0)
            dirc = ((miota * (2 * jj)) & kk) != 0
            swap = sw ^ dirc
            nka = jnp.where(swap, kb, ka)
            nkb = jnp.where(swap, ka, kb)
            nia = jnp.where(swap, ib, ia)
            nib = jnp.where(swap, ia, ib)
            nva = jnp.where(swap, vb, va)
            nvb = jnp.where(swap, va, vb)
            keys = jnp.concatenate(
                [nka[:, None], nkb[:, None]], axis=1).reshape(n, WB)
            idx = jnp.concatenate(
                [nia[:, None], nib[:, None]], axis=1).reshape(n, WB)
            kv = jnp.concatenate(
                [nva[:, None], nvb[:, None]], axis=1).reshape(n, WB)
        else:
            riota = jax.lax.broadcasted_iota(jnp.int32, (n, 1), 0)
            bitj = (riota & jj) != 0
            cmask = bitj ^ ((riota & kk) != 0)
            kp = jnp.where(bitj, pltpu.roll(keys, jj, 0),
                           pltpu.roll(keys, n - jj, 0))
            ip = jnp.where(bitj, pltpu.roll(idx, jj, 0),
                           pltpu.roll(idx, n - jj, 0))
            vp = jnp.where(bitj, pltpu.roll(kv, jj, 0),
                           pltpu.roll(kv, n - jj, 0))
            s1 = (kp > keys) | ((kp == keys) & (ip < idx))
            swap = s1 ^ cmask
            keys = jnp.where(swap, kp, keys)
            idx = jnp.where(swap, ip, idx)
            kv = jnp.where(swap, vp, kv)
        return keys, idx, kv

    for kk in [2 << t for t in range(9)]:
        jj = kk // 2
        while jj >= 1:
            keys, idx, kv = _stage(keys, idx, kv, kk, jj, NSORT)
            jj //= 2
    keys, idx, kv = _stage(keys, idx, kv, 2 * NSORT, NSORT // 2, NSORT)
    keys, idx, kv = keys[:NSORT // 2], idx[:NSORT // 2], kv[:NSORT // 2]
    jj = NSORT // 4
    while jj >= 1:
        keys, idx, kv = _stage(keys, idx, kv, 2 * NSORT, jj, NSORT // 2)
        jj //= 2

    rtop = jax.lax.bitcast_convert_type(keys[:TOPK], jnp.float32)
    ktop = kv[:TOPK]
    sv = jnp.concatenate([rtop, ktop], axis=0)

    y = jax.lax.dot_general(
        wm_ref[...], sv.astype(jnp.bfloat16),
        (((1,), (0,)), ((), ())),
        preferred_element_type=jnp.float32)
    o_ref[0, :, :] = y


def kernel(x, Wr, Wk, Wc, bc):
    rb = lambda a: a.astype(jnp.bfloat16)
    xp = rb(jnp.pad(jnp.moveaxis(x[0], 0, 1),
                    ((1, 1), (0, 0), (1, 1))))
    wr2 = rb(jnp.transpose(Wr.reshape(C, K * K, K, K),
                           (1, 0, 2, 3)).reshape(NCH, K * K))
    wk2 = rb(jnp.transpose(Wk.reshape(C, K * K, K, K),
                           (1, 0, 2, 3)).reshape(NCH, K * K))
    wm16 = Wc[:, :, 0, 0].astype(jnp.bfloat16)

    rowspec = lambda off: pl.BlockSpec(
        (1, C, W + 2), lambda i, off=off: (RB * i + off, 0, 0))
    y = pl.pallas_call(
        _row_kernel,
        grid=(H // RB,),
        in_specs=[
            rowspec(0), rowspec(1), rowspec(2),
            rowspec(3), rowspec(4), rowspec(5),
            pl.BlockSpec((NCH, K * K), lambda i: (0, 0)),
            pl.BlockSpec((NCH, K * K), lambda i: (0, 0)),
            pl.BlockSpec((C, 2 * TOPK), lambda i: (0, 0)),
        ],
        out_specs=pl.BlockSpec((1, C, WB), lambda i: (i, 0, 0)),
        out_shape=jax.ShapeDtypeStruct((H // RB, C, WB), jnp.float32),
    )(xp, xp, xp, xp, xp, xp, wr2, wk2, wm16)
    y = y.reshape(H // RB, C, RB, W).transpose(1, 0, 2, 3).reshape(C, H, W)
    return (y + bc[:, None, None])[None]
